# baseline (speedup 1.0000x reference)
import functools

import jax
import jax.numpy as jnp
from jax import lax
from jax.experimental import pallas as pl
from jax.experimental.pallas import tpu as pltpu
from jax.experimental.pallas import tpu_sc as plsc

H = 128
EPS = 1e-3
NC, NS = 2, 16
NW = NC * NS
LANES = 16
CPR = H // LANES

_GATHER_DNUMS = lax.GatherDimensionNumbers(
    offset_dims=(), collapsed_slice_dims=(0,), start_index_map=(0,))


def _splat_lane(x, u):
    idx = jnp.full((LANES, 1), u, jnp.int32)
    return lax.gather(x, idx, _GATHER_DNUMS, slice_sizes=(1,),
                      mode=lax.GatherScatterMode.PROMISE_IN_BOUNDS)


def _make_sc_gather_sum(N, L, rpw):
    ipc = rpw // 128
    ngr = rpw // LANES
    mesh = plsc.VectorSubcoreMesh(core_axis_name="c", subcore_axis_name="s")

    @functools.partial(
        pl.kernel,
        mesh=mesh,
        out_type=jax.ShapeDtypeStruct((N, H), jnp.float32),
        scratch_types=[
            pltpu.VMEM((ipc, 128), jnp.int32),
            pltpu.VMEM((ngr, LANES), jnp.int32),
            pltpu.VMEM((rpw, H), jnp.float32),
            pltpu.VMEM((rpw, H), jnp.float32),
            pltpu.VMEM((2, H), jnp.float32),
            pltpu.SemaphoreType.DMA,
        ],
    )
    def sc_kernel(tok_hbm, seg_hbm, wemb_hbm, semb_hbm, pemb_hbm, out_hbm,
                  tok_v, seg_v, rows_v, pos_v, semb_v, sem_w):
        cid = lax.axis_index("c")
        sid = lax.axis_index("s")
        wid = sid * NC + cid
        base = wid * rpw

        pltpu.sync_copy(tok_hbm.at[pl.ds(wid * ipc, ipc)], tok_v)
        pltpu.sync_copy(seg_hbm.at[pl.ds(wid * ngr, ngr)], seg_v)
        pltpu.sync_copy(semb_hbm, semb_v)

        copies = []
        for j in range(ipc):
            copies.append(pltpu.async_copy(
                wemb_hbm.at[tok_v.at[j]], rows_v.at[pl.ds(j * 128, 128)],
                sem_w))

        pltpu.sync_copy(pemb_hbm.at[pl.ds(lax.rem(base, L), rpw)], pos_v)
        for c in copies:
            c.wait()

        seg0 = [semb_v[0, pl.ds(c * LANES, LANES)] for c in range(CPR)]
        segd = [semb_v[1, pl.ds(c * LANES, LANES)] - seg0[c]
                for c in range(CPR)]

        def group_body(g, carry):
            sidxf = seg_v[g, :].astype(jnp.float32)
            for u in range(LANES):
                sf = _splat_lane(sidxf, u)
                r = g * LANES + u
                for c in range(CPR):
                    sl = pl.ds(c * LANES, LANES)
                    rows_v[r, sl] = (rows_v[r, sl] + pos_v[r, sl]
                                     + (seg0[c] + sf * segd[c]))
            return carry

        lax.fori_loop(0, ngr, group_body, 0)

        pltpu.sync_copy(rows_v, out_hbm.at[pl.ds(base, rpw)])

    return sc_kernel


def _ln_body(x_ref, gam_ref, bet_ref, o_ref):
    x = x_ref[...]
    mean = jnp.mean(x, axis=-1, keepdims=True)
    xc = x - mean
    var = jnp.mean(xc * xc, axis=-1, keepdims=True)
    o_ref[...] = xc * lax.rsqrt(var + EPS) * gam_ref[...] + bet_ref[...]


def _tc_layernorm(x, gamma, beta, bm):
    n = x.shape[0]
    return pl.pallas_call(
        _ln_body,
        grid=(n // bm,),
        in_specs=[
            pl.BlockSpec((bm, H), lambda i: (i, 0)),
            pl.BlockSpec((1, H), lambda i: (0, 0)),
            pl.BlockSpec((1, H), lambda i: (0, 0)),
        ],
        out_specs=pl.BlockSpec((bm, H), lambda i: (i, 0)),
        out_shape=jax.ShapeDtypeStruct((n, H), jnp.float32),
    )(x, gamma.reshape(1, H), beta.reshape(1, H))


def kernel(token, segment, word_emb, seg_emb, pos_emb, gamma, beta):
    B, L = token.shape
    N = B * L
    rpw = N // NW
    tok = token.reshape(N // 128, 128).astype(jnp.int32)
    seg = segment.reshape(N // LANES, LANES).astype(jnp.int32)
    summed = _make_sc_gather_sum(N, L, rpw)(
        tok, seg, word_emb, seg_emb, pos_emb)
    out = _tc_layernorm(summed, gamma, beta, bm=1024)
    return out.reshape(B, L, H)
